# baseline (speedup 1.0000x reference)
import functools
import math

import jax
import jax.numpy as jnp
import ml_dtypes
import numpy as np
from jax import lax
from jax.experimental import pallas as pl
from jax.experimental.pallas import tpu as pltpu
from jax.experimental.pallas import tpu_sc as plsc

D_MODEL = 128
MAX_SEQ_LEN = 2048
SCALE = math.sqrt(float(D_MODEL))

_NUM_CORES = 2
_NUM_SUBCORES = 16
_NW = _NUM_CORES * _NUM_SUBCORES
_BATCH = 4
_CH = MAX_SEQ_LEN // _NW
_BPW = _BATCH * _CH


def _pos_encoding(max_seq_len, d_model):
    position = np.arange(0, max_seq_len, dtype=np.float32)[:, None]
    div_term = np.exp(
        np.arange(0, d_model, 2, dtype=np.float32)
        * -(math.log(10000.0) / d_model)
    )
    enc = np.zeros((max_seq_len, d_model), dtype=np.float32)
    enc[:, 0::2] = np.sin(position * div_term).astype(np.float32)
    enc[:, 1::2] = np.cos(position * div_term).astype(np.float32)
    return enc


def _interleave_columns(pe):
    n, d = pe.shape
    v = pe.reshape(n, d // 32, 2, 16)
    return np.swapaxes(v, 2, 3).reshape(n, d)


_PE_PACKED = np.ascontiguousarray(
    _interleave_columns(_pos_encoding(MAX_SEQ_LEN, D_MODEL)).astype(
        ml_dtypes.bfloat16
    )
).view(np.int32)

_MESH = plsc.VectorSubcoreMesh(core_axis_name="c", subcore_axis_name="s")


@functools.partial(
    pl.kernel,
    out_type=jax.ShapeDtypeStruct((_BATCH, MAX_SEQ_LEN, D_MODEL), jnp.float32),
    mesh=_MESH,
    scratch_types=[
        pltpu.VMEM((_BATCH, _CH), jnp.int32),
        pltpu.VMEM((_BPW, D_MODEL), jnp.float32),
        pltpu.VMEM((_CH, D_MODEL // 2), jnp.int32),
        pltpu.SemaphoreType.DMA,
        pltpu.SemaphoreType.DMA,
        pltpu.SemaphoreType.DMA,
        pltpu.SemaphoreType.DMA,
    ],
)
def _emb_kernel(x_hbm, table_hbm, pe_hbm, out_hbm, idx_v, rows_v,
                pe_v, sem_i, sem_g, sem_p, sem_o):
    wid = lax.axis_index("s") * _NUM_CORES + lax.axis_index("c")
    pos_base = wid * _CH
    def _idx_copy(b):
        return pltpu.make_async_copy(
            x_hbm.at[b, pl.ds(pos_base, _CH)], idx_v.at[b], sem_i
        )

    def fire_idx(b, carry):
        _idx_copy(b).start()
        return carry

    lax.fori_loop(0, _BATCH, fire_idx, 0)
    p_cp = pltpu.async_copy(pe_hbm.at[pl.ds(pos_base, _CH)], pe_v, sem_p)

    def _gather(b):
        return pltpu.make_async_copy(
            table_hbm.at[idx_v.at[b]], rows_v.at[pl.ds(b * _CH, _CH)], sem_g
        )

    def _writeback(b):
        return pltpu.make_async_copy(
            rows_v.at[pl.ds(b * _CH, _CH)],
            out_hbm.at[b, pl.ds(pos_base, _CH)],
            sem_o,
        )

    def fire_gather(b, carry):
        _idx_copy(b).wait()
        _gather(b).start()
        return carry

    lax.fori_loop(0, _BATCH, fire_gather, 0)
    p_cp.wait()

    def fuse_chunk(b, carry):
        _gather(b).wait()

        @plsc.parallel_loop(0, _CH, unroll=2)
        def body(i):
            r = b * _CH + i
            for g in range(D_MODEL // 32):
                iw = pe_v[i, pl.ds(16 * g, 16)]
                lo = lax.bitcast_convert_type(iw << 16, jnp.float32)
                hi = lax.bitcast_convert_type(iw & np.int32(-65536), jnp.float32)
                for h, pe16 in enumerate((lo, hi)):
                    sl = pl.ds(32 * g + 16 * h, 16)
                    rows_v[r, sl] = rows_v[r, sl] * SCALE + pe16

        _writeback(b).start()
        return carry

    lax.fori_loop(0, _BATCH, fuse_chunk, 0)

    def drain(b, carry):
        _writeback(b).wait()
        return carry

    lax.fori_loop(0, _BATCH, drain, 0)


def kernel(x, table):
    pe_arr = jnp.concatenate(
        [jnp.asarray(_PE_PACKED[:1024]), jnp.asarray(_PE_PACKED[1024:])], axis=0
    )
    return _emb_kernel(x.astype(jnp.int32), jnp.asarray(table), pe_arr)


# speedup vs baseline: 1.0053x; 1.0053x over previous
import functools
import math

import jax
import jax.numpy as jnp
import ml_dtypes
import numpy as np
from jax import lax
from jax.experimental import pallas as pl
from jax.experimental.pallas import tpu as pltpu
from jax.experimental.pallas import tpu_sc as plsc

D_MODEL = 128
MAX_SEQ_LEN = 2048
SCALE = math.sqrt(float(D_MODEL))

_NUM_CORES = 2
_NUM_SUBCORES = 16
_NW = _NUM_CORES * _NUM_SUBCORES
_BATCH = 4
_CH = MAX_SEQ_LEN // _NW
_BPW = _BATCH * _CH


def _pos_encoding(max_seq_len, d_model):
    position = np.arange(0, max_seq_len, dtype=np.float32)[:, None]
    div_term = np.exp(
        np.arange(0, d_model, 2, dtype=np.float32)
        * -(math.log(10000.0) / d_model)
    )
    enc = np.zeros((max_seq_len, d_model), dtype=np.float32)
    enc[:, 0::2] = np.sin(position * div_term).astype(np.float32)
    enc[:, 1::2] = np.cos(position * div_term).astype(np.float32)
    return enc


def _interleave_columns(pe):
    n, d = pe.shape
    v = pe.reshape(n, d // 32, 2, 16)
    return np.swapaxes(v, 2, 3).reshape(n, d)


_PE_PACKED = np.ascontiguousarray(
    _interleave_columns(_pos_encoding(MAX_SEQ_LEN, D_MODEL)).astype(
        ml_dtypes.bfloat16
    )
).view(np.int32)

_MESH = plsc.VectorSubcoreMesh(core_axis_name="c", subcore_axis_name="s")


@functools.partial(
    pl.kernel,
    out_type=jax.ShapeDtypeStruct((_BATCH, MAX_SEQ_LEN, D_MODEL), jnp.float32),
    mesh=_MESH,
    scratch_types=[
        pltpu.VMEM((_BATCH, _CH), jnp.int32),
        pltpu.VMEM((_BPW, D_MODEL), jnp.float32),
        pltpu.VMEM((_CH, D_MODEL // 2), jnp.int32),
        pltpu.SemaphoreType.DMA,
        pltpu.SemaphoreType.DMA,
        pltpu.SemaphoreType.DMA,
        pltpu.SemaphoreType.DMA,
    ],
)
def _emb_kernel(x_hbm, table_hbm, pe_hbm, out_hbm, idx_v, rows_v,
                pe_v, sem_i, sem_g, sem_p, sem_o):
    wid = lax.axis_index("s") * _NUM_CORES + lax.axis_index("c")
    pos_base = wid * _CH
    def _idx_copy(b):
        return pltpu.make_async_copy(
            x_hbm.at[b, pl.ds(pos_base, _CH)], idx_v.at[b], sem_i
        )

    def fire_idx(b, carry):
        _idx_copy(b).start()
        return carry

    lax.fori_loop(0, _BATCH, fire_idx, 0)
    p_cp = pltpu.async_copy(pe_hbm.at[pl.ds(pos_base, _CH)], pe_v, sem_p)

    def _gather(b):
        return pltpu.make_async_copy(
            table_hbm.at[idx_v.at[b]], rows_v.at[pl.ds(b * _CH, _CH)], sem_g
        )

    def _writeback(b):
        return pltpu.make_async_copy(
            rows_v.at[pl.ds(b * _CH, _CH)],
            out_hbm.at[b, pl.ds(pos_base, _CH)],
            sem_o,
        )

    def fire_gather(b, carry):
        _idx_copy(b).wait()
        _gather(b).start()
        return carry

    lax.fori_loop(0, _BATCH, fire_gather, 0)
    p_cp.wait()

    def fuse_chunk(b, carry):
        _gather(b).wait()

        @plsc.parallel_loop(0, _CH, unroll=2)
        def body(i):
            r = b * _CH + i
            for g in range(D_MODEL // 32):
                iw = pe_v[i, pl.ds(16 * g, 16)]
                lo = lax.bitcast_convert_type(iw << 16, jnp.float32)
                hi = lax.bitcast_convert_type(iw & np.int32(-65536), jnp.float32)
                for h, pe16 in enumerate((lo, hi)):
                    sl = pl.ds(32 * g + 16 * h, 16)
                    rows_v[r, sl] = rows_v[r, sl] * SCALE + pe16

        _writeback(b).start()
        return carry

    lax.fori_loop(0, _BATCH, fuse_chunk, 0)

    def drain(b, carry):
        _writeback(b).wait()
        return carry

    lax.fori_loop(0, _BATCH, drain, 0)


def kernel(x, table):
    return _emb_kernel(
        x.astype(jnp.int32), jnp.asarray(table), jnp.asarray(_PE_PACKED)
    )
